# Initial kernel scaffold; baseline (speedup 1.0000x reference)
import jax
import jax.numpy as jnp
from jax.experimental import pallas as pl


def kernel(x, table):
    raise NotImplementedError("write your pallas kernel here")


# speedup vs baseline: 5.4745x; 5.4745x over previous
import functools
import math

import jax
import jax.numpy as jnp
from jax import lax
from jax.experimental import pallas as pl
from jax.experimental.pallas import tpu as pltpu
from jax.experimental.pallas import tpu_sc as plsc

D = 128
SCALE = math.sqrt(D)

NC = 2
NS = 16
NW = NC * NS

CH = 128


def _scale_body(t_ref, o_ref):
    o_ref[...] = t_ref[...] * SCALE


def _scale_table(table):
    v, d = table.shape
    blk = 1000
    return pl.pallas_call(
        _scale_body,
        grid=(v // blk,),
        in_specs=[pl.BlockSpec((blk, d), lambda i: (i, 0))],
        out_specs=pl.BlockSpec((blk, d), lambda i: (i, 0)),
        out_shape=jax.ShapeDtypeStruct((v, d), table.dtype),
    )(table)


@functools.partial(jax.jit, static_argnames=("b_total",))
def _sc_gather(x_grp, table_scaled, b_total):
    g_per_w = x_grp.shape[1]
    b_per_w = g_per_w * CH

    mesh = plsc.VectorSubcoreMesh(core_axis_name="c", subcore_axis_name="s")

    @functools.partial(
        pl.kernel,
        mesh=mesh,
        out_type=jax.ShapeDtypeStruct((b_total, D), jnp.float32),
        scratch_types=[
            pltpu.VMEM((g_per_w, CH), jnp.int32),
            pltpu.VMEM((CH, D), jnp.float32),
            pltpu.SemaphoreType.DMA,
        ],
    )
    def k(x_hbm, tab_hbm, out_hbm, idx_v, rows_v, gsem):
        wid = lax.axis_index("s") * NC + lax.axis_index("c")
        base = wid * b_per_w
        pltpu.sync_copy(x_hbm.at[wid], idx_v)

        def chunk(g, _):
            pltpu.async_copy(tab_hbm.at[idx_v.at[g]], rows_v, gsem).wait()
            pltpu.sync_copy(rows_v, out_hbm.at[pl.ds(base + g * CH, CH)])
            return 0

        lax.fori_loop(0, g_per_w, chunk, 0)

    return k(x_grp, table_scaled)


def kernel(x, table):
    bs, sl = x.shape
    b_total = bs * sl
    x_grp = x.reshape(NW, b_total // (NW * CH), CH).astype(jnp.int32)
    out = _sc_gather(x_grp, _scale_table(table), b_total)
    return out.reshape(bs, sl, D)


# speedup vs baseline: 7.5429x; 1.3778x over previous
import functools
import math

import jax
import jax.numpy as jnp
from jax import lax
from jax.experimental import pallas as pl
from jax.experimental.pallas import tpu as pltpu
from jax.experimental.pallas import tpu_sc as plsc

D = 128
SCALE = math.sqrt(D)

NC = 2
NS = 16
NW = NC * NS

CH = 128


def _scale_body(t_ref, o_ref):
    o_ref[...] = t_ref[...] * SCALE


def _scale_table(table):
    v, d = table.shape
    blk = 1000
    return pl.pallas_call(
        _scale_body,
        grid=(v // blk,),
        in_specs=[pl.BlockSpec((blk, d), lambda i: (i, 0))],
        out_specs=pl.BlockSpec((blk, d), lambda i: (i, 0)),
        out_shape=jax.ShapeDtypeStruct((v, d), table.dtype),
    )(table)


NBUF = 4


@functools.partial(jax.jit, static_argnames=("b_total",))
def _sc_gather(x_grp, table_scaled, b_total):
    g_per_w = x_grp.shape[1]
    b_per_w = g_per_w * CH
    niter = g_per_w // NBUF

    mesh = plsc.VectorSubcoreMesh(core_axis_name="c", subcore_axis_name="s")

    @functools.partial(
        pl.kernel,
        mesh=mesh,
        out_type=jax.ShapeDtypeStruct((b_total, D), jnp.float32),
        scratch_types=[
            pltpu.VMEM((g_per_w, CH), jnp.int32),
            pltpu.VMEM((NBUF, CH, D), jnp.float32),
        ]
        + [pltpu.SemaphoreType.DMA] * (2 * NBUF),
    )
    def k(x_hbm, tab_hbm, out_hbm, idx_v, bufs, *sems):
        gsems, osems = sems[:NBUF], sems[NBUF:]
        wid = lax.axis_index("s") * NC + lax.axis_index("c")
        base = wid * b_per_w
        pltpu.sync_copy(x_hbm.at[wid], idx_v)

        def g_issue(g, b):
            pltpu.async_copy(tab_hbm.at[idx_v.at[g]], bufs.at[b], gsems[b])

        def g_wait(g, b):
            pltpu.make_async_copy(
                tab_hbm.at[idx_v.at[g]], bufs.at[b], gsems[b]
            ).wait()

        def s_issue(g, b):
            pltpu.async_copy(
                bufs.at[b], out_hbm.at[pl.ds(base + g * CH, CH)], osems[b]
            )

        def s_wait(g, b):
            pltpu.make_async_copy(
                bufs.at[b], out_hbm.at[pl.ds(base + g * CH, CH)], osems[b]
            ).wait()

        for b in range(NBUF):
            g_issue(b, b)
        for b in range(NBUF - 2):
            g_wait(b, b)
            s_issue(b, b)

        def outer(i, _):
            for b in range(NBUF):
                t = i * NBUF + b
                s_wait(t - NBUF, b)
                g_issue(t, b)
                g_wait(t - 2, (b + 2) % NBUF)
                s_issue(t - 2, (b + 2) % NBUF)
            return 0

        lax.fori_loop(1, niter, outer, 0)

        gl = g_per_w
        for t in (gl - 2, gl - 1):
            b = t % NBUF
            g_wait(t, b)
            s_issue(t, b)
        for t in range(gl - NBUF, gl):
            s_wait(t, t % NBUF)

    return k(x_grp, table_scaled)


def kernel(x, table):
    bs, sl = x.shape
    b_total = bs * sl
    x_grp = x.reshape(NW, b_total // (NW * CH), CH).astype(jnp.int32)
    out = _sc_gather(x_grp, _scale_table(table), b_total)
    return out.reshape(bs, sl, D)


# speedup vs baseline: 8.3235x; 1.1035x over previous
import functools
import math

import jax
import jax.numpy as jnp
from jax import lax
from jax.experimental import pallas as pl
from jax.experimental.pallas import tpu as pltpu
from jax.experimental.pallas import tpu_sc as plsc

D = 128
SCALE = math.sqrt(D)

NC = 2
NS = 16
NW = NC * NS

CH = 128


def _scale_body(t_ref, o_ref):
    o_ref[...] = t_ref[...] * SCALE


def _scale_table(table):
    v, d = table.shape
    blk = 5000
    return pl.pallas_call(
        _scale_body,
        grid=(v // blk,),
        in_specs=[pl.BlockSpec((blk, d), lambda i: (i, 0))],
        out_specs=pl.BlockSpec((blk, d), lambda i: (i, 0)),
        out_shape=jax.ShapeDtypeStruct((v, d), table.dtype),
    )(table)


NBUF = 4


@functools.partial(jax.jit, static_argnames=("b_total",))
def _sc_gather(x_grp, table_scaled, b_total):
    g_per_w = x_grp.shape[1]
    b_per_w = g_per_w * CH
    niter = g_per_w // NBUF

    mesh = plsc.VectorSubcoreMesh(core_axis_name="c", subcore_axis_name="s")

    @functools.partial(
        pl.kernel,
        mesh=mesh,
        out_type=jax.ShapeDtypeStruct((b_total, D), jnp.float32),
        scratch_types=[
            pltpu.VMEM((g_per_w, CH), jnp.int32),
            pltpu.VMEM((NBUF, CH, D), jnp.float32),
        ]
        + [pltpu.SemaphoreType.DMA] * (2 * NBUF),
    )
    def k(x_hbm, tab_hbm, out_hbm, idx_v, bufs, *sems):
        gsems, osems = sems[:NBUF], sems[NBUF:]
        wid = lax.axis_index("s") * NC + lax.axis_index("c")
        base = wid * b_per_w
        pltpu.sync_copy(x_hbm.at[wid], idx_v)

        def g_issue(g, b):
            pltpu.async_copy(tab_hbm.at[idx_v.at[g]], bufs.at[b], gsems[b])

        def g_wait(g, b):
            pltpu.make_async_copy(
                tab_hbm.at[idx_v.at[g]], bufs.at[b], gsems[b]
            ).wait()

        def s_issue(g, b):
            pltpu.async_copy(
                bufs.at[b], out_hbm.at[pl.ds(base + g * CH, CH)], osems[b]
            )

        def s_wait(g, b):
            pltpu.make_async_copy(
                bufs.at[b], out_hbm.at[pl.ds(base + g * CH, CH)], osems[b]
            ).wait()

        for b in range(NBUF):
            g_issue(b, b)
        for b in range(NBUF - 2):
            g_wait(b, b)
            s_issue(b, b)

        def outer(i, _):
            for b in range(NBUF):
                t = i * NBUF + b
                s_wait(t - NBUF, b)
                g_issue(t, b)
                g_wait(t - 2, (b + 2) % NBUF)
                s_issue(t - 2, (b + 2) % NBUF)
            return 0

        lax.fori_loop(1, niter, outer, 0)

        gl = g_per_w
        for t in (gl - 2, gl - 1):
            b = t % NBUF
            g_wait(t, b)
            s_issue(t, b)
        for t in range(gl - NBUF, gl):
            s_wait(t, t % NBUF)

    return k(x_grp, table_scaled)


def kernel(x, table):
    bs, sl = x.shape
    b_total = bs * sl
    x_grp = x.reshape(NW, b_total // (NW * CH), CH).astype(jnp.int32)
    out = _sc_gather(x_grp, _scale_table(table), b_total)
    return out.reshape(bs, sl, D)


# speedup vs baseline: 9.2351x; 1.1095x over previous
import functools
import math

import jax
import jax.numpy as jnp
from jax import lax
from jax.experimental import pallas as pl
from jax.experimental.pallas import tpu as pltpu
from jax.experimental.pallas import tpu_sc as plsc

D = 128
SCALE = math.sqrt(D)

NC = 2
NS = 16
NW = NC * NS

CH = 128
NBUF = 4
L = 16
RU = 2


@functools.partial(jax.jit, static_argnames=("b_total",))
def _sc_gather(x_grp, table, b_total):
    g_per_w = x_grp.shape[1]
    b_per_w = g_per_w * CH
    niter = g_per_w // NBUF

    mesh = plsc.VectorSubcoreMesh(core_axis_name="c", subcore_axis_name="s")

    @functools.partial(
        pl.kernel,
        mesh=mesh,
        out_type=jax.ShapeDtypeStruct((b_total, D), jnp.float32),
        scratch_types=[
            pltpu.VMEM((g_per_w, CH), jnp.int32),
            pltpu.VMEM((NBUF, CH, D), jnp.float32),
        ]
        + [pltpu.SemaphoreType.DMA] * (2 * NBUF),
    )
    def k(x_hbm, tab_hbm, out_hbm, idx_v, bufs, *sems):
        gsems, osems = sems[:NBUF], sems[NBUF:]
        wid = lax.axis_index("s") * NC + lax.axis_index("c")
        base = wid * b_per_w
        pltpu.sync_copy(x_hbm.at[wid], idx_v)

        def g_issue(g, b):
            pltpu.async_copy(tab_hbm.at[idx_v.at[g]], bufs.at[b], gsems[b])

        def g_wait(g, b):
            pltpu.make_async_copy(
                tab_hbm.at[idx_v.at[g]], bufs.at[b], gsems[b]
            ).wait()

        def s_issue(g, b):
            pltpu.async_copy(
                bufs.at[b], out_hbm.at[pl.ds(base + g * CH, CH)], osems[b]
            )

        def s_wait(g, b):
            pltpu.make_async_copy(
                bufs.at[b], out_hbm.at[pl.ds(base + g * CH, CH)], osems[b]
            ).wait()

        def scale_buf(b):
            def sbody(r, _):
                for u in range(RU):
                    for c in range(D // L):
                        sl = pl.ds(c * L, L)
                        bufs[b, r * RU + u, sl] = bufs[b, r * RU + u, sl] * SCALE
                return 0

            lax.fori_loop(0, CH // RU, sbody, 0, unroll=2)

        for b in range(NBUF):
            g_issue(b, b)
        for b in range(NBUF - 2):
            g_wait(b, b)
            scale_buf(b)
            s_issue(b, b)

        def outer(i, _):
            for b in range(NBUF):
                t = i * NBUF + b
                s_wait(t - NBUF, b)
                g_issue(t, b)
                g_wait(t - 2, (b + 2) % NBUF)
                scale_buf((b + 2) % NBUF)
                s_issue(t - 2, (b + 2) % NBUF)
            return 0

        lax.fori_loop(1, niter, outer, 0)

        gl = g_per_w
        for t in (gl - 2, gl - 1):
            b = t % NBUF
            g_wait(t, b)
            scale_buf(b)
            s_issue(t, b)
        for t in range(gl - NBUF, gl):
            s_wait(t, t % NBUF)

    return k(x_grp, table)


def kernel(x, table):
    bs, sl = x.shape
    b_total = bs * sl
    x_grp = x.reshape(NW, b_total // (NW * CH), CH).astype(jnp.int32)
    out = _sc_gather(x_grp, table, b_total)
    return out.reshape(bs, sl, D)


# speedup vs baseline: 9.2389x; 1.0004x over previous
import functools
import math

import jax
import jax.numpy as jnp
from jax import lax
from jax.experimental import pallas as pl
from jax.experimental.pallas import tpu as pltpu
from jax.experimental.pallas import tpu_sc as plsc

D = 128
SCALE = math.sqrt(D)

NC = 2
NS = 16
NW = NC * NS

CH = 128
NBUF = 5
LAG = 3
L = 16
RU = 2


@functools.partial(jax.jit, static_argnames=("b_total",))
def _sc_gather(x_grp, table, b_total):
    g_per_w = x_grp.shape[1]
    b_per_w = g_per_w * CH
    niter = g_per_w // NBUF

    mesh = plsc.VectorSubcoreMesh(core_axis_name="c", subcore_axis_name="s")

    @functools.partial(
        pl.kernel,
        mesh=mesh,
        out_type=jax.ShapeDtypeStruct((b_total, D), jnp.float32),
        scratch_types=[
            pltpu.VMEM((g_per_w, CH), jnp.int32),
            pltpu.VMEM((NBUF, CH, D), jnp.float32),
        ]
        + [pltpu.SemaphoreType.DMA] * (2 * NBUF),
    )
    def k(x_hbm, tab_hbm, out_hbm, idx_v, bufs, *sems):
        gsems, osems = sems[:NBUF], sems[NBUF:]
        wid = lax.axis_index("s") * NC + lax.axis_index("c")
        base = wid * b_per_w
        pltpu.sync_copy(x_hbm.at[wid], idx_v)

        def g_issue(g, b):
            pltpu.async_copy(tab_hbm.at[idx_v.at[g]], bufs.at[b], gsems[b])

        def g_wait(g, b):
            pltpu.make_async_copy(
                tab_hbm.at[idx_v.at[g]], bufs.at[b], gsems[b]
            ).wait()

        def s_issue(g, b):
            pltpu.async_copy(
                bufs.at[b], out_hbm.at[pl.ds(base + g * CH, CH)], osems[b]
            )

        def s_wait(g, b):
            pltpu.make_async_copy(
                bufs.at[b], out_hbm.at[pl.ds(base + g * CH, CH)], osems[b]
            ).wait()

        def scale_buf(b):
            def sbody(r, _):
                for u in range(RU):
                    for c in range(D // L):
                        sl = pl.ds(c * L, L)
                        bufs[b, r * RU + u, sl] = bufs[b, r * RU + u, sl] * SCALE
                return 0

            lax.fori_loop(0, CH // RU, sbody, 0, unroll=2)

        for b in range(NBUF):
            g_issue(b, b)
        for b in range(NBUF - LAG):
            g_wait(b, b)
            scale_buf(b)
            s_issue(b, b)

        def outer(i, _):
            for b in range(NBUF):
                t = i * NBUF + b
                s_wait(t - NBUF, b)
                g_issue(t, b)
                bl = (b - LAG) % NBUF
                g_wait(t - LAG, bl)
                scale_buf(bl)
                s_issue(t - LAG, bl)
            return 0

        lax.fori_loop(1, niter, outer, 0)

        gl = g_per_w
        for t in range(gl - LAG, gl):
            b = t % NBUF
            g_wait(t, b)
            scale_buf(b)
            s_issue(t, b)
        for t in range(gl - NBUF, gl):
            s_wait(t, t % NBUF)

    return k(x_grp, table)


def kernel(x, table):
    bs, sl = x.shape
    b_total = bs * sl
    x_grp = x.reshape(NW, b_total // (NW * CH), CH).astype(jnp.int32)
    out = _sc_gather(x_grp, table, b_total)
    return out.reshape(bs, sl, D)
